# baseline (speedup 1.0000x reference)
import math

import numpy as np
import jax
import jax.numpy as jnp
from jax import lax
from jax.experimental import pallas as pl
from jax.experimental.pallas import tpu as pltpu
from jax.experimental.pallas import tpu_sc as plsc

DIM = 1024
NSUB = 512
KDIM = 256
CDIM = 256

TOKENS = 8192
TB = 256
NBLK = TOKENS // TB


def _stages(n):
    out = []
    k = 2
    while k <= n:
        j = k // 2
        while j >= 1:
            out.append((k, j))
            j //= 2
        k *= 2
    return out

_ST128 = _stages(128)

_PAIRS = [(r, s) for r in range(32) for s in range(32) if (r + 1) * (s + 1) <= 32]
_NPAIR = len(_PAIRS)
_R_TAB = np.array([p[0] for p in _PAIRS] + [0] * (128 - _NPAIR), np.int32)
_S_TAB = np.array([p[1] for p in _PAIRS] + [0] * (128 - _NPAIR), np.int32)


def _bitonic_desc_packed(K):
    lane = lax.broadcasted_iota(jnp.int32, (TB, 128), 1)
    for k, j in _ST128:
        perm = lane ^ j
        asc = (lane & k) != 0
        tm = ((lane & j) != 0) == asc
        P = jnp.take_along_axis(K, perm, axis=-1)
        K = jnp.where(tm, jnp.maximum(K, P), jnp.minimum(K, P))
    return K


def _mono(f):
    b = lax.bitcast_convert_type(f, jnp.int32)
    return jnp.where(b >= 0, b, b ^ jnp.int32(0x7FFFFFFF))


def _bitonic_desc_kv(X, ID):
    lane = lax.broadcasted_iota(jnp.int32, (TB, 128), 1)
    for k, j in _ST128:
        perm = lane ^ j
        asc = (lane & k) != 0
        tm = ((lane & j) != 0) == asc
        P = jnp.take_along_axis(X, perm, axis=-1)
        Pid = jnp.take_along_axis(ID, perm, axis=-1)
        win = (P > X) | ((P == X) & (Pid < ID))
        take = win == tm
        X = jnp.where(take, P, X)
        ID = jnp.where(take, Pid, ID)
    return X, ID


def _side_top32(S):
    V = [S[:, c * 128:(c + 1) * 128] for c in range(4)]
    M = jnp.maximum(jnp.maximum(V[0], V[1]), jnp.maximum(V[2], V[3]))
    lane = lax.broadcasted_iota(jnp.int32, (TB, 128), 1)
    keyM = (_mono(M) & ~jnp.int32(0x7F)) | (127 - lane)
    sM = _bitonic_desc_packed(keyM)
    bids = 127 - (sM[:, :32] & jnp.int32(0x7F))
    gs, oi = [], []
    for c in range(4):
        gs.append(jnp.take_along_axis(V[c], bids, axis=-1))
        oi.append(bids + c * 128)
    cv, cid = _bitonic_desc_kv(jnp.concatenate(gs, axis=-1),
                               jnp.concatenate(oi, axis=-1))
    return cv[:, :32], cid[:, :32]


def _select_body(x_ref, wq_ref, ka_ref, kb_ref, rtab_ref, stab_ref,
                 fidx_ref, w_ref, ssum_ref, smax_ref):
    i = pl.program_id(0)
    q = jnp.dot(x_ref[:], wq_ref[:], preferred_element_type=jnp.float32)
    qa = q[:, :KDIM]
    qb = q[:, KDIM:]
    sa = jnp.dot(qa, ka_ref[:], preferred_element_type=jnp.float32)
    sb = jnp.dot(qb, kb_ref[:], preferred_element_type=jnp.float32)

    va, ia = _side_top32(sa)
    vb, ib = _side_top32(sb)

    rt = jnp.broadcast_to(rtab_ref[0:1, :], (TB, 128))
    st = jnp.broadcast_to(stab_ref[0:1, :], (TB, 128))
    stair = (jnp.take_along_axis(va, rt, axis=-1)
             + jnp.take_along_axis(vb, st, axis=-1))
    pidx = lax.broadcasted_iota(jnp.int32, (TB, 128), 1)
    stair = jnp.where(pidx < _NPAIR, stair, -jnp.inf)
    gv, gid = _bitonic_desc_kv(stair, pidx)
    fs = gv[:, :32]
    p = gid[:, :32]
    r = jnp.take_along_axis(rt, p, axis=-1)
    s = jnp.take_along_axis(st, p, axis=-1)
    fia = jnp.take_along_axis(ia, r, axis=-1)
    fib = jnp.take_along_axis(ib, s, axis=-1)
    fi = fia * NSUB + fib

    inv_t = 1.0 / math.sqrt(2.0 * KDIM)
    e = jnp.exp((fs - fs[:, 0:1]) * inv_t)
    w = e / jnp.sum(e, axis=-1, keepdims=True)

    col5 = lax.broadcasted_iota(jnp.int32, (32, 512), 1)
    row5 = lax.broadcasted_iota(jnp.int32, (32, 512), 0)
    E_rep = (col5 // 16 == row5).astype(jnp.float32)
    w_rep = jnp.dot(w, E_rep, preferred_element_type=jnp.float32)

    fidx_ref[:] = fi
    w_ref[:] = w_rep

    bsum = jnp.sum(fs)
    bmax = jnp.max(fs)

    @pl.when(i == 0)
    def _():
        ssum_ref[0, 0] = bsum
        smax_ref[0, 0] = bmax

    @pl.when(i > 0)
    def _():
        ssum_ref[0, 0] = ssum_ref[0, 0] + bsum
        smax_ref[0, 0] = jnp.maximum(smax_ref[0, 0], bmax)


def _run_select(x2, W_q, kaT, kbT, ntok):
    return pl.pallas_call(
        _select_body,
        grid=(ntok // TB,),
        in_specs=[
            pl.BlockSpec((TB, DIM), lambda i: (i, 0)),
            pl.BlockSpec((DIM, 2 * KDIM), lambda i: (0, 0)),
            pl.BlockSpec((KDIM, NSUB), lambda i: (0, 0)),
            pl.BlockSpec((KDIM, NSUB), lambda i: (0, 0)),
            pl.BlockSpec((8, 128), lambda i: (0, 0)),
            pl.BlockSpec((8, 128), lambda i: (0, 0)),
        ],
        out_specs=[
            pl.BlockSpec((TB, 32), lambda i: (i, 0)),
            pl.BlockSpec((TB, 512), lambda i: (i, 0)),
            pl.BlockSpec(memory_space=pltpu.SMEM),
            pl.BlockSpec(memory_space=pltpu.SMEM),
        ],
        out_shape=[
            jax.ShapeDtypeStruct((ntok, 32), jnp.int32),
            jax.ShapeDtypeStruct((ntok, 512), jnp.float32),
            jax.ShapeDtypeStruct((1, 1), jnp.float32),
            jax.ShapeDtypeStruct((1, 1), jnp.float32),
        ],
        compiler_params=pltpu.CompilerParams(
            dimension_semantics=("arbitrary",)),
    )(x2, W_q, kaT, kbT,
      jnp.broadcast_to(jnp.asarray(_R_TAB)[None, :], (8, 128)),
      jnp.broadcast_to(jnp.asarray(_S_TAB)[None, :], (8, 128)))


_NC = 2
_NS = 16
_NW = _NC * _NS
_CH = 4


def _gather_body(ntok, codes_hbm, fidx_hbm, wts_hbm, out_hbm,
                 idx_v, w_v, rows_v, out_v, sem0, sem1):
    tpw = ntok // _NW
    nstep = tpw // _CH
    wid = lax.axis_index("s") * _NC + lax.axis_index("c")
    tok0 = wid * tpw
    sems = (sem0, sem1)

    def stage(j, b):
        base = (tok0 + j * _CH) * 32
        pltpu.sync_copy(fidx_hbm.at[pl.ds(base, _CH * 32)],
                        idx_v.at[b])
        pltpu.sync_copy(wts_hbm.at[pl.ds(base, _CH * 32)],
                        w_v.at[b])
        pltpu.async_copy(codes_hbm.at[idx_v.at[b]], rows_v.at[b], sems[b])

    def compute(j, b):
        pltpu.make_async_copy(codes_hbm.at[idx_v.at[b]],
                              rows_v.at[b], sems[b]).wait()
        for t in range(_CH):
            def kbody(k, accs):
                r = t * 32 + k
                wk = w_v[b, r, pl.ds(0, 16)]
                return tuple(accs[d] + wk * rows_v[b, r, pl.ds(d * 16, 16)]
                             for d in range(16))
            accs = lax.fori_loop(
                0, 32, kbody,
                tuple(jnp.zeros((16,), jnp.float32) for _ in range(16)),
                unroll=2)
            for d in range(16):
                out_v[t, pl.ds(d * 16, 16)] = accs[d]
        pltpu.sync_copy(out_v, out_hbm.at[pl.ds(tok0 + j * _CH, _CH)])

    stage(0, 0)

    def two_steps(j0, carry):
        stage(j0 + 1, 1)
        compute(j0, 0)
        stage(jnp.minimum(j0 + 2, nstep - 1), 0)
        compute(j0 + 1, 1)
        return carry

    lax.fori_loop(0, nstep // 2, lambda i, c: two_steps(2 * i, c), 0)
    pltpu.make_async_copy(codes_hbm.at[idx_v.at[0]], rows_v.at[0],
                          sems[0]).wait()


import functools


@functools.lru_cache(maxsize=None)
def _make_gather(ntok):
    return pl.kernel(
        functools.partial(_gather_body, ntok),
        out_type=jax.ShapeDtypeStruct((ntok, CDIM), jnp.float32),
        mesh=plsc.VectorSubcoreMesh(core_axis_name="c", subcore_axis_name="s",
                                    num_cores=_NC, num_subcores=_NS),
        scratch_types=[
            pltpu.VMEM((2, _CH * 32), jnp.int32),
            pltpu.VMEM((2, _CH * 32, 16), jnp.float32),
            pltpu.VMEM((2, _CH * 32, CDIM), jnp.float32),
            pltpu.VMEM((_CH, CDIM), jnp.float32),
            pltpu.SemaphoreType.DMA,
            pltpu.SemaphoreType.DMA,
        ],
    )


def _mlp_body(m_ref, w1_ref, w2_ref, o_ref):
    h = jnp.dot(m_ref[:], w1_ref[:], preferred_element_type=jnp.float32)
    h = h / (1.0 + jnp.exp(-h))
    o_ref[:] = jnp.dot(h, w2_ref[:], preferred_element_type=jnp.float32)


def _run_mlp(mixed, W1, W2, ntok):
    return pl.pallas_call(
        _mlp_body,
        grid=(ntok // TB,),
        in_specs=[
            pl.BlockSpec((TB, CDIM), lambda i: (i, 0)),
            pl.BlockSpec((CDIM, DIM), lambda i: (0, 0)),
            pl.BlockSpec((DIM, DIM), lambda i: (0, 0)),
        ],
        out_specs=pl.BlockSpec((TB, DIM), lambda i: (i, 0)),
        out_shape=jax.ShapeDtypeStruct((ntok, DIM), jnp.float32),
        compiler_params=pltpu.CompilerParams(
            dimension_semantics=("arbitrary",)),
    )(mixed, W1, W2)


def kernel(x, W_q, key_a, key_b, codes, W1, W2):
    batch, seq, _ = x.shape
    x2 = x.reshape(batch * seq, DIM)
    nchunk = 8
    ct = TOKENS // nchunk
    kaT, kbT = key_a.T, key_b.T
    gather = _make_gather(ct)
    sel = []
    for c in range(nchunk):
        sel.append(_run_select(x2[c * ct:(c + 1) * ct], W_q, kaT, kbT, ct))
    gs = []
    for c in range(nchunk):
        f, wr, _, _ = sel[c]
        gs.append(gather(codes, f.reshape(-1), wr.reshape(ct * 32, 16)))
    ys = [_run_mlp(g, W1, W2, ct) for g in gs]
    out = jnp.concatenate(ys, axis=0).reshape(batch, seq, DIM)
    ssum = sum(s[2][0, 0] for s in sel)
    smax = sel[0][3][0, 0]
    for c in range(1, nchunk):
        smax = jnp.maximum(smax, sel[c][3][0, 0])
    stats_mean = ssum / float(TOKENS * 32)
    stats_max = smax
    return (out, stats_mean, stats_max)


# speedup vs baseline: 1.0604x; 1.0604x over previous
import math

import numpy as np
import jax
import jax.numpy as jnp
from jax import lax
from jax.experimental import pallas as pl
from jax.experimental.pallas import tpu as pltpu
from jax.experimental.pallas import tpu_sc as plsc

DIM = 1024
NSUB = 512
KDIM = 256
CDIM = 256

TOKENS = 8192
TB = 512
NBLK = TOKENS // TB


def _stages(n):
    out = []
    k = 2
    while k <= n:
        j = k // 2
        while j >= 1:
            out.append((k, j))
            j //= 2
        k *= 2
    return out

_ST128 = _stages(128)

_PAIRS = [(r, s) for r in range(32) for s in range(32) if (r + 1) * (s + 1) <= 32]
_NPAIR = len(_PAIRS)
_R_TAB = np.array([p[0] for p in _PAIRS] + [0] * (128 - _NPAIR), np.int32)
_S_TAB = np.array([p[1] for p in _PAIRS] + [0] * (128 - _NPAIR), np.int32)


def _bitonic_desc_packed(K):
    lane = lax.broadcasted_iota(jnp.int32, (TB, 128), 1)
    for k, j in _ST128:
        perm = lane ^ j
        asc = (lane & k) != 0
        tm = ((lane & j) != 0) == asc
        P = jnp.take_along_axis(K, perm, axis=-1)
        K = jnp.where(tm, jnp.maximum(K, P), jnp.minimum(K, P))
    return K


def _mono(f):
    b = lax.bitcast_convert_type(f, jnp.int32)
    return jnp.where(b >= 0, b, b ^ jnp.int32(0x7FFFFFFF))


def _bitonic_desc_kv(X, ID):
    lane = lax.broadcasted_iota(jnp.int32, (TB, 128), 1)
    for k, j in _ST128:
        perm = lane ^ j
        asc = (lane & k) != 0
        tm = ((lane & j) != 0) == asc
        P = jnp.take_along_axis(X, perm, axis=-1)
        Pid = jnp.take_along_axis(ID, perm, axis=-1)
        win = (P > X) | ((P == X) & (Pid < ID))
        take = win == tm
        X = jnp.where(take, P, X)
        ID = jnp.where(take, Pid, ID)
    return X, ID


def _side_top32(S):
    V = [S[:, c * 128:(c + 1) * 128] for c in range(4)]
    M = jnp.maximum(jnp.maximum(V[0], V[1]), jnp.maximum(V[2], V[3]))
    lane = lax.broadcasted_iota(jnp.int32, (TB, 128), 1)
    keyM = (_mono(M) & ~jnp.int32(0x7F)) | (127 - lane)
    sM = _bitonic_desc_packed(keyM)
    bids = 127 - (sM[:, :32] & jnp.int32(0x7F))
    gs, oi = [], []
    for c in range(4):
        gs.append(jnp.take_along_axis(V[c], bids, axis=-1))
        oi.append(bids + c * 128)
    cv, cid = _bitonic_desc_kv(jnp.concatenate(gs, axis=-1),
                               jnp.concatenate(oi, axis=-1))
    return cv[:, :32], cid[:, :32]


def _select_body(x_ref, wq_ref, ka_ref, kb_ref, rtab_ref, stab_ref,
                 fidx_ref, w_ref, ssum_ref, smax_ref):
    i = pl.program_id(0)
    q = jnp.dot(x_ref[:], wq_ref[:], preferred_element_type=jnp.float32)
    qa = q[:, :KDIM]
    qb = q[:, KDIM:]
    sa = jnp.dot(qa, ka_ref[:], preferred_element_type=jnp.float32)
    sb = jnp.dot(qb, kb_ref[:], preferred_element_type=jnp.float32)

    va, ia = _side_top32(sa)
    vb, ib = _side_top32(sb)

    rt = jnp.broadcast_to(rtab_ref[0:1, :], (TB, 128))
    st = jnp.broadcast_to(stab_ref[0:1, :], (TB, 128))
    stair = (jnp.take_along_axis(va, rt, axis=-1)
             + jnp.take_along_axis(vb, st, axis=-1))
    pidx = lax.broadcasted_iota(jnp.int32, (TB, 128), 1)
    stair = jnp.where(pidx < _NPAIR, stair, -jnp.inf)
    gv, gid = _bitonic_desc_kv(stair, pidx)
    fs = gv[:, :32]
    p = gid[:, :32]
    r = jnp.take_along_axis(rt, p, axis=-1)
    s = jnp.take_along_axis(st, p, axis=-1)
    fia = jnp.take_along_axis(ia, r, axis=-1)
    fib = jnp.take_along_axis(ib, s, axis=-1)
    fi = fia * NSUB + fib

    inv_t = 1.0 / math.sqrt(2.0 * KDIM)
    e = jnp.exp((fs - fs[:, 0:1]) * inv_t)
    w = e / jnp.sum(e, axis=-1, keepdims=True)

    col5 = lax.broadcasted_iota(jnp.int32, (32, 512), 1)
    row5 = lax.broadcasted_iota(jnp.int32, (32, 512), 0)
    E_rep = (col5 // 16 == row5).astype(jnp.float32)
    w_rep = jnp.dot(w, E_rep, preferred_element_type=jnp.float32)

    fidx_ref[:] = fi
    w_ref[:] = w_rep

    bsum = jnp.sum(fs)
    bmax = jnp.max(fs)

    @pl.when(i == 0)
    def _():
        ssum_ref[0, 0] = bsum
        smax_ref[0, 0] = bmax

    @pl.when(i > 0)
    def _():
        ssum_ref[0, 0] = ssum_ref[0, 0] + bsum
        smax_ref[0, 0] = jnp.maximum(smax_ref[0, 0], bmax)


def _run_select(x2, W_q, kaT, kbT, ntok):
    return pl.pallas_call(
        _select_body,
        grid=(ntok // TB,),
        in_specs=[
            pl.BlockSpec((TB, DIM), lambda i: (i, 0)),
            pl.BlockSpec((DIM, 2 * KDIM), lambda i: (0, 0)),
            pl.BlockSpec((KDIM, NSUB), lambda i: (0, 0)),
            pl.BlockSpec((KDIM, NSUB), lambda i: (0, 0)),
            pl.BlockSpec((8, 128), lambda i: (0, 0)),
            pl.BlockSpec((8, 128), lambda i: (0, 0)),
        ],
        out_specs=[
            pl.BlockSpec((TB, 32), lambda i: (i, 0)),
            pl.BlockSpec((TB, 512), lambda i: (i, 0)),
            pl.BlockSpec(memory_space=pltpu.SMEM),
            pl.BlockSpec(memory_space=pltpu.SMEM),
        ],
        out_shape=[
            jax.ShapeDtypeStruct((ntok, 32), jnp.int32),
            jax.ShapeDtypeStruct((ntok, 512), jnp.float32),
            jax.ShapeDtypeStruct((1, 1), jnp.float32),
            jax.ShapeDtypeStruct((1, 1), jnp.float32),
        ],
        compiler_params=pltpu.CompilerParams(
            dimension_semantics=("arbitrary",)),
    )(x2, W_q, kaT, kbT,
      jnp.broadcast_to(jnp.asarray(_R_TAB)[None, :], (8, 128)),
      jnp.broadcast_to(jnp.asarray(_S_TAB)[None, :], (8, 128)))


_NC = 2
_NS = 16
_NW = _NC * _NS
_CH = 4


def _gather_body(ntok, codes_hbm, fidx_hbm, wts_hbm, out_hbm,
                 idx_v, w_v, rows_v, out_v, sem0, sem1):
    tpw = ntok // _NW
    nstep = tpw // _CH
    wid = lax.axis_index("s") * _NC + lax.axis_index("c")
    tok0 = wid * tpw
    sems = (sem0, sem1)

    def stage(j, b):
        base = (tok0 + j * _CH) * 32
        pltpu.sync_copy(fidx_hbm.at[pl.ds(base, _CH * 32)],
                        idx_v.at[b])
        pltpu.sync_copy(wts_hbm.at[pl.ds(base, _CH * 32)],
                        w_v.at[b])
        pltpu.async_copy(codes_hbm.at[idx_v.at[b]], rows_v.at[b], sems[b])

    def compute(j, b):
        pltpu.make_async_copy(codes_hbm.at[idx_v.at[b]],
                              rows_v.at[b], sems[b]).wait()
        for t in range(_CH):
            def kbody(k, accs):
                r = t * 32 + k
                wk = w_v[b, r, pl.ds(0, 16)]
                return tuple(accs[d] + wk * rows_v[b, r, pl.ds(d * 16, 16)]
                             for d in range(16))
            accs = lax.fori_loop(
                0, 32, kbody,
                tuple(jnp.zeros((16,), jnp.float32) for _ in range(16)),
                unroll=2)
            for d in range(16):
                out_v[t, pl.ds(d * 16, 16)] = accs[d]
        pltpu.sync_copy(out_v, out_hbm.at[pl.ds(tok0 + j * _CH, _CH)])

    stage(0, 0)

    def two_steps(j0, carry):
        stage(j0 + 1, 1)
        compute(j0, 0)
        stage(jnp.minimum(j0 + 2, nstep - 1), 0)
        compute(j0 + 1, 1)
        return carry

    lax.fori_loop(0, nstep // 2, lambda i, c: two_steps(2 * i, c), 0)
    pltpu.make_async_copy(codes_hbm.at[idx_v.at[0]], rows_v.at[0],
                          sems[0]).wait()


import functools


@functools.lru_cache(maxsize=None)
def _make_gather(ntok):
    return pl.kernel(
        functools.partial(_gather_body, ntok),
        out_type=jax.ShapeDtypeStruct((ntok, CDIM), jnp.float32),
        mesh=plsc.VectorSubcoreMesh(core_axis_name="c", subcore_axis_name="s",
                                    num_cores=_NC, num_subcores=_NS),
        scratch_types=[
            pltpu.VMEM((2, _CH * 32), jnp.int32),
            pltpu.VMEM((2, _CH * 32, 16), jnp.float32),
            pltpu.VMEM((2, _CH * 32, CDIM), jnp.float32),
            pltpu.VMEM((_CH, CDIM), jnp.float32),
            pltpu.SemaphoreType.DMA,
            pltpu.SemaphoreType.DMA,
        ],
    )


def _mlp_body(m_ref, w1_ref, w2_ref, o_ref):
    h = jnp.dot(m_ref[:], w1_ref[:], preferred_element_type=jnp.float32)
    h = h / (1.0 + jnp.exp(-h))
    o_ref[:] = jnp.dot(h, w2_ref[:], preferred_element_type=jnp.float32)


def _run_mlp(mixed, W1, W2, ntok):
    return pl.pallas_call(
        _mlp_body,
        grid=(ntok // TB,),
        in_specs=[
            pl.BlockSpec((TB, CDIM), lambda i: (i, 0)),
            pl.BlockSpec((CDIM, DIM), lambda i: (0, 0)),
            pl.BlockSpec((DIM, DIM), lambda i: (0, 0)),
        ],
        out_specs=pl.BlockSpec((TB, DIM), lambda i: (i, 0)),
        out_shape=jax.ShapeDtypeStruct((ntok, DIM), jnp.float32),
        compiler_params=pltpu.CompilerParams(
            dimension_semantics=("arbitrary",)),
    )(mixed, W1, W2)


def kernel(x, W_q, key_a, key_b, codes, W1, W2):
    batch, seq, _ = x.shape
    x2 = x.reshape(batch * seq, DIM)
    nchunk = 4
    ct = TOKENS // nchunk
    kaT, kbT = key_a.T, key_b.T
    gather = _make_gather(ct)
    sel = []
    for c in range(nchunk):
        sel.append(_run_select(x2[c * ct:(c + 1) * ct], W_q, kaT, kbT, ct))
    gs = []
    for c in range(nchunk):
        f, wr, _, _ = sel[c]
        gs.append(gather(codes, f.reshape(-1), wr.reshape(ct * 32, 16)))
    ys = [_run_mlp(g, W1, W2, ct) for g in gs]
    out = jnp.concatenate(ys, axis=0).reshape(batch, seq, DIM)
    ssum = sum(s[2][0, 0] for s in sel)
    smax = sel[0][3][0, 0]
    for c in range(1, nchunk):
        smax = jnp.maximum(smax, sel[c][3][0, 0])
    stats_mean = ssum / float(TOKENS * 32)
    stats_max = smax
    return (out, stats_mean, stats_max)


# speedup vs baseline: 1.0927x; 1.0304x over previous
import math

import numpy as np
import jax
import jax.numpy as jnp
from jax import lax
from jax.experimental import pallas as pl
from jax.experimental.pallas import tpu as pltpu
from jax.experimental.pallas import tpu_sc as plsc

DIM = 1024
NSUB = 512
KDIM = 256
CDIM = 256

TOKENS = 8192
TB = 128
NBLK = TOKENS // TB


def _stages(n):
    out = []
    k = 2
    while k <= n:
        j = k // 2
        while j >= 1:
            out.append((k, j))
            j //= 2
        k *= 2
    return out

_ST128 = _stages(128)

_PAIRS = [(r, s) for r in range(32) for s in range(32) if (r + 1) * (s + 1) <= 32]
_NPAIR = len(_PAIRS)
_R_TAB = np.array([p[0] for p in _PAIRS] + [0] * (128 - _NPAIR), np.int32)
_S_TAB = np.array([p[1] for p in _PAIRS] + [0] * (128 - _NPAIR), np.int32)


def _bitonic_desc_packed(K):
    lane = lax.broadcasted_iota(jnp.int32, (TB, 128), 1)
    for k, j in _ST128:
        perm = lane ^ j
        asc = (lane & k) != 0
        tm = ((lane & j) != 0) == asc
        P = jnp.take_along_axis(K, perm, axis=-1)
        K = jnp.where(tm, jnp.maximum(K, P), jnp.minimum(K, P))
    return K


def _mono(f):
    b = lax.bitcast_convert_type(f, jnp.int32)
    return jnp.where(b >= 0, b, b ^ jnp.int32(0x7FFFFFFF))


def _bitonic_desc_kv(X, ID):
    lane = lax.broadcasted_iota(jnp.int32, (TB, 128), 1)
    for k, j in _ST128:
        perm = lane ^ j
        asc = (lane & k) != 0
        tm = ((lane & j) != 0) == asc
        P = jnp.take_along_axis(X, perm, axis=-1)
        Pid = jnp.take_along_axis(ID, perm, axis=-1)
        win = (P > X) | ((P == X) & (Pid < ID))
        take = win == tm
        X = jnp.where(take, P, X)
        ID = jnp.where(take, Pid, ID)
    return X, ID


def _side_top32(S):
    V = [S[:, c * 128:(c + 1) * 128] for c in range(4)]
    M = jnp.maximum(jnp.maximum(V[0], V[1]), jnp.maximum(V[2], V[3]))
    lane = lax.broadcasted_iota(jnp.int32, (TB, 128), 1)
    keyM = (_mono(M) & ~jnp.int32(0x7F)) | (127 - lane)
    sM = _bitonic_desc_packed(keyM)
    bids = 127 - (sM[:, :32] & jnp.int32(0x7F))
    gs, oi = [], []
    for c in range(4):
        gs.append(jnp.take_along_axis(V[c], bids, axis=-1))
        oi.append(bids + c * 128)
    cv, cid = _bitonic_desc_kv(jnp.concatenate(gs, axis=-1),
                               jnp.concatenate(oi, axis=-1))
    return cv[:, :32], cid[:, :32]


def _select_body(x_ref, wq_ref, ka_ref, kb_ref, rtab_ref, stab_ref,
                 fidx_ref, w_ref, ssum_ref, smax_ref):
    i = pl.program_id(0)
    q = jnp.dot(x_ref[:], wq_ref[:], preferred_element_type=jnp.float32)
    qa = q[:, :KDIM]
    qb = q[:, KDIM:]
    sa = jnp.dot(qa, ka_ref[:], preferred_element_type=jnp.float32)
    sb = jnp.dot(qb, kb_ref[:], preferred_element_type=jnp.float32)

    va, ia = _side_top32(sa)
    vb, ib = _side_top32(sb)

    rt = jnp.broadcast_to(rtab_ref[0:1, :], (TB, 128))
    st = jnp.broadcast_to(stab_ref[0:1, :], (TB, 128))
    stair = (jnp.take_along_axis(va, rt, axis=-1)
             + jnp.take_along_axis(vb, st, axis=-1))
    pidx = lax.broadcasted_iota(jnp.int32, (TB, 128), 1)
    stair = jnp.where(pidx < _NPAIR, stair, -jnp.inf)
    gv, gid = _bitonic_desc_kv(stair, pidx)
    fs = gv[:, :32]
    p = gid[:, :32]
    r = jnp.take_along_axis(rt, p, axis=-1)
    s = jnp.take_along_axis(st, p, axis=-1)
    fia = jnp.take_along_axis(ia, r, axis=-1)
    fib = jnp.take_along_axis(ib, s, axis=-1)
    fi = fia * NSUB + fib

    inv_t = 1.0 / math.sqrt(2.0 * KDIM)
    e = jnp.exp((fs - fs[:, 0:1]) * inv_t)
    w = e / jnp.sum(e, axis=-1, keepdims=True)

    col5 = lax.broadcasted_iota(jnp.int32, (32, 512), 1)
    row5 = lax.broadcasted_iota(jnp.int32, (32, 512), 0)
    E_rep = (col5 // 16 == row5).astype(jnp.float32)
    w_rep = jnp.dot(w, E_rep, preferred_element_type=jnp.float32)

    fidx_ref[:] = fi
    w_ref[:] = w_rep

    bsum = jnp.sum(fs)
    bmax = jnp.max(fs)

    @pl.when(i == 0)
    def _():
        ssum_ref[0, 0] = bsum
        smax_ref[0, 0] = bmax

    @pl.when(i > 0)
    def _():
        ssum_ref[0, 0] = ssum_ref[0, 0] + bsum
        smax_ref[0, 0] = jnp.maximum(smax_ref[0, 0], bmax)


def _run_select(x2, W_q, kaT, kbT, ntok):
    return pl.pallas_call(
        _select_body,
        grid=(ntok // TB,),
        in_specs=[
            pl.BlockSpec((TB, DIM), lambda i: (i, 0)),
            pl.BlockSpec((DIM, 2 * KDIM), lambda i: (0, 0)),
            pl.BlockSpec((KDIM, NSUB), lambda i: (0, 0)),
            pl.BlockSpec((KDIM, NSUB), lambda i: (0, 0)),
            pl.BlockSpec((8, 128), lambda i: (0, 0)),
            pl.BlockSpec((8, 128), lambda i: (0, 0)),
        ],
        out_specs=[
            pl.BlockSpec((TB, 32), lambda i: (i, 0)),
            pl.BlockSpec((TB, 512), lambda i: (i, 0)),
            pl.BlockSpec(memory_space=pltpu.SMEM),
            pl.BlockSpec(memory_space=pltpu.SMEM),
        ],
        out_shape=[
            jax.ShapeDtypeStruct((ntok, 32), jnp.int32),
            jax.ShapeDtypeStruct((ntok, 512), jnp.float32),
            jax.ShapeDtypeStruct((1, 1), jnp.float32),
            jax.ShapeDtypeStruct((1, 1), jnp.float32),
        ],
        compiler_params=pltpu.CompilerParams(
            dimension_semantics=("arbitrary",)),
    )(x2, W_q, kaT, kbT,
      jnp.broadcast_to(jnp.asarray(_R_TAB)[None, :], (8, 128)),
      jnp.broadcast_to(jnp.asarray(_S_TAB)[None, :], (8, 128)))


_NC = 2
_NS = 16
_NW = _NC * _NS
_CH = 4


def _gather_body(ntok, codes_hbm, fidx_hbm, wts_hbm, out_hbm,
                 idx_v, w_v, rows_v, out_v, sem0, sem1):
    tpw = ntok // _NW
    nstep = tpw // _CH
    wid = lax.axis_index("s") * _NC + lax.axis_index("c")
    tok0 = wid * tpw
    sems = (sem0, sem1)

    def stage(j, b):
        base = (tok0 + j * _CH) * 32
        pltpu.sync_copy(fidx_hbm.at[pl.ds(base, _CH * 32)],
                        idx_v.at[b])
        pltpu.sync_copy(wts_hbm.at[pl.ds(base, _CH * 32)],
                        w_v.at[b])
        pltpu.async_copy(codes_hbm.at[idx_v.at[b]], rows_v.at[b], sems[b])

    def compute(j, b):
        pltpu.make_async_copy(codes_hbm.at[idx_v.at[b]],
                              rows_v.at[b], sems[b]).wait()
        for t in range(_CH):
            def kbody(k, accs):
                r = t * 32 + k
                wk = w_v[b, r, pl.ds(0, 16)]
                return tuple(accs[d] + wk * rows_v[b, r, pl.ds(d * 16, 16)]
                             for d in range(16))
            accs = lax.fori_loop(
                0, 32, kbody,
                tuple(jnp.zeros((16,), jnp.float32) for _ in range(16)),
                unroll=2)
            for d in range(16):
                out_v[t, pl.ds(d * 16, 16)] = accs[d]
        pltpu.sync_copy(out_v, out_hbm.at[pl.ds(tok0 + j * _CH, _CH)])

    stage(0, 0)

    def two_steps(j0, carry):
        stage(j0 + 1, 1)
        compute(j0, 0)
        stage(jnp.minimum(j0 + 2, nstep - 1), 0)
        compute(j0 + 1, 1)
        return carry

    lax.fori_loop(0, nstep // 2, lambda i, c: two_steps(2 * i, c), 0)
    pltpu.make_async_copy(codes_hbm.at[idx_v.at[0]], rows_v.at[0],
                          sems[0]).wait()


import functools


@functools.lru_cache(maxsize=None)
def _make_gather(ntok):
    return pl.kernel(
        functools.partial(_gather_body, ntok),
        out_type=jax.ShapeDtypeStruct((ntok, CDIM), jnp.float32),
        mesh=plsc.VectorSubcoreMesh(core_axis_name="c", subcore_axis_name="s",
                                    num_cores=_NC, num_subcores=_NS),
        scratch_types=[
            pltpu.VMEM((2, _CH * 32), jnp.int32),
            pltpu.VMEM((2, _CH * 32, 16), jnp.float32),
            pltpu.VMEM((2, _CH * 32, CDIM), jnp.float32),
            pltpu.VMEM((_CH, CDIM), jnp.float32),
            pltpu.SemaphoreType.DMA,
            pltpu.SemaphoreType.DMA,
        ],
    )


def _mlp_body(m_ref, w1_ref, w2_ref, o_ref):
    h = jnp.dot(m_ref[:], w1_ref[:], preferred_element_type=jnp.float32)
    h = h / (1.0 + jnp.exp(-h))
    o_ref[:] = jnp.dot(h, w2_ref[:], preferred_element_type=jnp.float32)


def _run_mlp(mixed, W1, W2, ntok):
    return pl.pallas_call(
        _mlp_body,
        grid=(ntok // TB,),
        in_specs=[
            pl.BlockSpec((TB, CDIM), lambda i: (i, 0)),
            pl.BlockSpec((CDIM, DIM), lambda i: (0, 0)),
            pl.BlockSpec((DIM, DIM), lambda i: (0, 0)),
        ],
        out_specs=pl.BlockSpec((TB, DIM), lambda i: (i, 0)),
        out_shape=jax.ShapeDtypeStruct((ntok, DIM), jnp.float32),
        compiler_params=pltpu.CompilerParams(
            dimension_semantics=("arbitrary",)),
    )(mixed, W1, W2)


def kernel(x, W_q, key_a, key_b, codes, W1, W2):
    batch, seq, _ = x.shape
    x2 = x.reshape(batch * seq, DIM)
    nchunk = 4
    ct = TOKENS // nchunk
    kaT, kbT = key_a.T, key_b.T
    gather = _make_gather(ct)
    sel = []
    for c in range(nchunk):
        sel.append(_run_select(x2[c * ct:(c + 1) * ct], W_q, kaT, kbT, ct))
    gs = []
    for c in range(nchunk):
        f, wr, _, _ = sel[c]
        gs.append(gather(codes, f.reshape(-1), wr.reshape(ct * 32, 16)))
    ys = [_run_mlp(g, W1, W2, ct) for g in gs]
    out = jnp.concatenate(ys, axis=0).reshape(batch, seq, DIM)
    ssum = sum(s[2][0, 0] for s in sel)
    smax = sel[0][3][0, 0]
    for c in range(1, nchunk):
        smax = jnp.maximum(smax, sel[c][3][0, 0])
    stats_mean = ssum / float(TOKENS * 32)
    stats_max = smax
    return (out, stats_mean, stats_max)
